# baseline (speedup 1.0000x reference)
import jax
import jax.numpy as jnp
from jax import lax
from jax.experimental import pallas as pl
from jax.experimental.pallas import tpu as pltpu, tpu_sc as plsc

B, K, A, V = 4096, 26, 100, 128
NC, NS = 2, 16
NW = NC * NS
SPW = B // NW
LANES = 16
G = 2
GK = G * K
GA = G * A
TRASH = GA
NG = SPW // G
NBUF = 2


def _winner_sweep(iv0, iv1):
    w0 = jnp.zeros((LANES,), jnp.int32)
    w1 = jnp.zeros((LANES,), jnp.int32)
    idx_sc = [iv0[k] if k < LANES else iv1[k - (K - LANES)]
              for k in range(K)]
    for kp in range(K):
        kv = jnp.full((LANES,), kp, jnp.int32)
        w0 = jnp.where(iv0 == idx_sc[kp], kv, w0)
        w1 = jnp.where(iv1 == idx_sc[kp], kv, w1)
    return w0, w1


def _body(values_hbm, idx_hbm, out_hbm, idx_v, zero_v, vals_v, sidx_v, row_v,
          sz0, sz1, so0, so1, sv0, sv1, sin0, sin1):
    c = lax.axis_index("c")
    s = lax.axis_index("s")
    wid = s * NC + c
    base = wid * SPW

    pltpu.sync_copy(idx_hbm.at[pl.ds(base, SPW)], idx_v)

    zero16 = jnp.zeros((LANES,), jnp.float32)

    def zsrc(a, acc):
        for j in range(V // LANES):
            zero_v[a, pl.ds(j * LANES, LANES)] = zero16
        return acc
    lax.fori_loop(0, GK, zsrc, 0)

    for d0 in range(NBUF):
        for i in range((GA + 8) // GK):
            pltpu.sync_copy(zero_v, row_v.at[s, d0, pl.ds(i * GK, GK)])

    sem_zs = (sz0, sz1)
    sem_out = (so0, so1)
    sem_vs = (sv0, sv1)
    sem_in = (sin0, sin1)

    l0 = lax.iota(jnp.int32, LANES)
    l1 = l0 + (K - LANES)

    def in_cp(slot, q):
        return pltpu.make_async_copy(
            values_hbm.at[pl.ds((base + q * 2 * G) * K, 2 * GK)],
            vals_v.at[slot], sem_in[slot])

    def zs_cp(d, p):
        return pltpu.make_async_copy(
            zero_v, row_v.at[s, d].at[sidx_v.at[d, p]], sem_zs[d])

    def vs_cp(d, p, slot):
        return pltpu.make_async_copy(
            vals_v.at[slot, pl.ds(d * GK, GK)],
            row_v.at[s, d].at[sidx_v.at[d, p]], sem_vs[d])

    def out_cp(d, gi):
        return pltpu.make_async_copy(
            row_v.at[s, d, pl.ds(0, GA)],
            out_hbm.at[pl.ds((base + gi * G) * A, GA)], sem_out[d])

    in_cp(0, 0).start()
    in_cp(1, 1).start()

    NQ = NG // 2

    def step(g, carry):
        for qq in range(2):
            q = 2 * g + qq
            p = qq
            in_cp(qq, q).wait()
            for d in range(NBUF):
                gi = NBUF * q + d

                @pl.when(gi >= 2)
                def _():
                    out_cp(d, gi - 2).wait()
                    zs_cp(d, 1 - p).start()

                for gs in range(G):
                    si = gi * G + gs
                    iv0 = idx_v[si, pl.ds(0, LANES)]
                    iv1 = idx_v[si, pl.ds(K - LANES, LANES)]
                    w0, w1 = _winner_sweep(iv0, iv1)
                    t0 = jnp.where(w0 == l0, iv0 + gs * A, TRASH)
                    t1 = jnp.where(w1 == l1, iv1 + gs * A, TRASH)
                    sidx_v[d, p, pl.ds(gs * K, LANES)] = t0
                    sidx_v[d, p, pl.ds(gs * K + K - LANES, LANES)] = t1

                @pl.when(gi >= 2)
                def _():
                    zs_cp(d, 1 - p).wait()

                vs_cp(d, p, qq).start()
                vs_cp(d, p, qq).wait()
                out_cp(d, gi).start()

            @pl.when(q + 2 < NQ)
            def _():
                in_cp(qq, q + 2).start()
        return carry

    lax.fori_loop(0, NQ // 2, step, 0)

    out_cp(0, NG - 2).wait()
    out_cp(1, NG - 1).wait()


def kernel(values, attrib_idx):
    idx32 = attrib_idx.astype(jnp.int32)
    values2 = values.reshape(B * K, V)
    mesh = plsc.VectorSubcoreMesh(core_axis_name="c", subcore_axis_name="s")
    run = pl.kernel(
        _body,
        out_type=jax.ShapeDtypeStruct((B * A, V), jnp.float32),
        mesh=mesh,
        scratch_types=[
            pltpu.VMEM((SPW, K), jnp.int32),
            pltpu.VMEM((GK, V), jnp.float32),
            pltpu.VMEM((2, 2 * GK, V), jnp.float32),
            pltpu.VMEM((NBUF, 2, GK), jnp.int32),
            pltpu.VMEM_SHARED((NS, NBUF, GA + 8, V), jnp.float32),
        ] + [pltpu.SemaphoreType.DMA] * 8,
    )
    out = run(values2, idx32)
    return out.reshape(B, A, V)
